# Initial kernel scaffold; baseline (speedup 1.0000x reference)
import jax
import jax.numpy as jnp
from jax.experimental import pallas as pl


def kernel(z_e, codebook):
    raise NotImplementedError("write your pallas kernel here")


# speedup vs baseline: 1.3787x; 1.3787x over previous
import functools

import jax
import jax.numpy as jnp
from jax import lax
from jax.experimental import pallas as pl
from jax.experimental.pallas import tpu as pltpu
from jax.experimental.pallas import tpu_sc as plsc

KK = 1024
DD = 64
BETA = 0.25
BN = 512

NC = 2
NS = 16
NW = NC * NS
GCHUNK = 128


def _dist_argmin_body(z_ref, et_ref, idx_ref, sse_ref):
    z = z_ref[...]
    et = et_ref[...]
    z2 = jnp.sum(z * z, axis=1, keepdims=True)
    e2 = jnp.sum(et * et, axis=0, keepdims=True)
    mm = lax.dot_general(z, et, (((1,), (0,)), ((), ())),
                         preferred_element_type=jnp.float32)
    d = (z2 + e2) - 2.0 * mm
    m = jnp.min(d, axis=1, keepdims=True)
    iota = lax.broadcasted_iota(jnp.int32, d.shape, 1)
    idx = jnp.min(jnp.where(d == m, iota, KK), axis=1, keepdims=True)
    idx_ref[...] = idx

    @pl.when(pl.program_id(0) == 0)
    def _():
        sse_ref[0, 0] = 0.0

    sse_ref[0, 0] += jnp.sum(m)


def _dist_argmin(z, et):
    n = z.shape[0]
    grid = n // BN
    return pl.pallas_call(
        _dist_argmin_body,
        grid=(grid,),
        in_specs=[
            pl.BlockSpec((BN, DD), lambda i: (i, 0)),
            pl.BlockSpec((DD, KK), lambda i: (0, 0)),
        ],
        out_specs=[
            pl.BlockSpec((BN, 1), lambda i: (i, 0)),
            pl.BlockSpec((1, 1), lambda i: (0, 0), memory_space=pltpu.SMEM),
        ],
        out_shape=[
            jax.ShapeDtypeStruct((n, 1), jnp.int32),
            jax.ShapeDtypeStruct((1, 1), jnp.float32),
        ],
    )(z, et)


def _gather_body(table_hbm, idx_hbm, out_hbm, idx_v, rows_v, sem):
    wid = lax.axis_index("s") * NC + lax.axis_index("c")
    rows_per_w = idx_v.shape[0]
    base = wid * rows_per_w
    pltpu.sync_copy(idx_hbm.at[pl.ds(base, rows_per_w)], idx_v)
    copies = [
        pltpu.async_copy(
            table_hbm.at[idx_v.at[pl.ds(c * GCHUNK, GCHUNK)]],
            rows_v.at[pl.ds(c * GCHUNK, GCHUNK)],
            sem,
        )
        for c in range(rows_per_w // GCHUNK)
    ]
    for cp in copies:
        cp.wait()
    pltpu.sync_copy(rows_v, out_hbm.at[pl.ds(base, rows_per_w)])


def _sc_gather(table, idx):
    n = idx.shape[0]
    rows_per_w = n // NW
    mesh = plsc.VectorSubcoreMesh(core_axis_name="c", subcore_axis_name="s")
    fn = pl.kernel(
        _gather_body,
        out_type=jax.ShapeDtypeStruct((n, DD), jnp.float32),
        mesh=mesh,
        scratch_types=[
            pltpu.VMEM((rows_per_w,), jnp.int32),
            pltpu.VMEM((rows_per_w, DD), jnp.float32),
            pltpu.SemaphoreType.DMA,
        ],
        compiler_params=pltpu.CompilerParams(use_tc_tiling_on_sc=False),
    )
    return fn(table, idx)


def kernel(z_e, codebook):
    b, c, h, w = z_e.shape
    n = b * h * w
    z = jnp.transpose(z_e, (0, 2, 3, 1)).reshape(n, c)
    idx2d, sse = _dist_argmin(z, codebook.T)
    idx = idx2d.reshape(n)
    z_q_flat = _sc_gather(codebook, idx)
    z_q = jnp.transpose(z_q_flat.reshape(b, h, w, c), (0, 3, 1, 2))
    commit = BETA * (sse[0, 0] / jnp.float32(n * c))
    z_q_out = z_e + (z_q - z_e)
    codebook_loss = jnp.zeros(())
    indices_out = idx.reshape(b, h, w)
    return (z_q_out, codebook_loss, commit, commit, indices_out)


# speedup vs baseline: 2.0628x; 1.4962x over previous
import functools

import jax
import jax.numpy as jnp
from jax import lax
from jax.experimental import pallas as pl
from jax.experimental.pallas import tpu as pltpu

KK = 1024
DD = 64
BETA = 0.25


def _vq_body(ze_ref, e_ref, et_ref, zq_ref, idx_ref, sse_ref):
    zb = ze_ref[0]
    e = e_ref[...]
    et = et_ref[...]
    z2 = jnp.sum(zb * zb, axis=0, keepdims=True)
    e2 = jnp.sum(e * e, axis=1, keepdims=True)
    mm = lax.dot_general(e, zb, (((1,), (0,)), ((), ())),
                         preferred_element_type=jnp.float32)
    d = (z2 + e2) - 2.0 * mm
    m = jnp.min(d, axis=0, keepdims=True)
    iota = lax.broadcasted_iota(jnp.int32, d.shape, 0)
    idx = jnp.min(jnp.where(d == m, iota, KK), axis=0, keepdims=True)
    onehot = jnp.where(iota == idx, 1.0, 0.0)
    zq = lax.dot_general(et, onehot, (((1,), (0,)), ((), ())),
                         preferred_element_type=jnp.float32)
    zq_ref[0] = zb + (zq - zb)
    idx_ref[0] = idx

    @pl.when(pl.program_id(0) == 0)
    def _():
        sse_ref[0, 0] = 0.0

    sse_ref[0, 0] += jnp.sum(m)


def _vq(ze3, e):
    b = ze3.shape[0]
    hw = ze3.shape[2]
    return pl.pallas_call(
        _vq_body,
        grid=(b,),
        in_specs=[
            pl.BlockSpec((1, DD, hw), lambda i: (i, 0, 0)),
            pl.BlockSpec((KK, DD), lambda i: (0, 0)),
            pl.BlockSpec((DD, KK), lambda i: (0, 0)),
        ],
        out_specs=[
            pl.BlockSpec((1, DD, hw), lambda i: (i, 0, 0)),
            pl.BlockSpec((1, 1, hw), lambda i: (i, 0, 0)),
            pl.BlockSpec((1, 1), lambda i: (0, 0), memory_space=pltpu.SMEM),
        ],
        out_shape=[
            jax.ShapeDtypeStruct((b, DD, hw), jnp.float32),
            jax.ShapeDtypeStruct((b, 1, hw), jnp.int32),
            jax.ShapeDtypeStruct((1, 1), jnp.float32),
        ],
    )(ze3, e, e.T)


def kernel(z_e, codebook):
    b, c, h, w = z_e.shape
    hw = h * w
    ze3 = z_e.reshape(b, c, hw)
    zq3, idx3, sse = _vq(ze3, codebook)
    commit = BETA * (sse[0, 0] / jnp.float32(b * c * hw))
    z_q_out = zq3.reshape(b, c, h, w)
    indices_out = idx3.reshape(b, h, w)
    codebook_loss = jnp.zeros(())
    return (z_q_out, codebook_loss, commit, commit, indices_out)
